# baseline (speedup 1.0000x reference)
import functools

import jax
import jax.numpy as jnp
from jax import lax
from jax.experimental import pallas as pl
from jax.experimental.pallas import tpu as pltpu
from jax.experimental.pallas import tpu_sc as plsc

N_ITEMS = 100000
N_USERS = 100000
K = 200
P = 50
B = 4096
L = 16

NC = 2
NS = 16
NW = NC * NS
QW = B // NW
CH = 32
NCH = QW // CH
TBL = N_ITEMS + L

MK = 13
MP = 4

K_OFFS = [16 * g for g in range(12)] + [K - L]
K_TAIL_FRESH = 8
P_OFFS = [0, 16, 32, P - L]
P_TAIL_FRESH = 14


def _knn_body(mk_hbm, shk_hbm, mp_hbm, shp_hbm,
              nidxm_hbm, nsimm_hbm, profm_hbm, out_hbm,
              tbl, sbuf, mk_v, mp_v, shk_v, shp_v,
              nidx_v, nsim_v, prof_v, score_v, sem):
    wid = lax.axis_index("s") * NC + lax.axis_index("c")
    base = wid * QW
    lane = lax.iota(jnp.int32, L)
    zeros = jnp.zeros((L,), jnp.float32)

    def zero_body(j, carry):
        tbl[pl.ds(j * L, L)] = zeros
        return carry
    lax.fori_loop(0, TBL // L, zero_body, 0)

    sbuf[pl.ds(0, L)] = jnp.full((L,), -1, jnp.int32)
    sbuf[pl.ds(2 * L, L)] = jnp.full((L,), -2, jnp.int32)

    def histo_add(pidx):
        sk = jnp.sort(pidx)
        sbuf[pl.ds(L, L)] = sk
        prev = sbuf[pl.ds(L - 1, L)]
        nxt = sbuf[pl.ds(L + 1, L)]
        isfirst = sk != prev
        islast = sk != nxt
        run_start = plsc.cummax(jnp.where(isfirst, lane, 0))
        addcnt = (lane - run_start + 1).astype(jnp.float32)
        prevcnt = plsc.load_gather(tbl, [sk])
        plsc.store_scatter(tbl, [sk], prevcnt + addcnt, mask=islast)

    def chunk_body(c, carry):
        off = base + c * CH
        pltpu.sync_copy(mk_hbm.at[pl.ds(off * MK, CH * MK)], mk_v)
        pltpu.sync_copy(mp_hbm.at[pl.ds(off * MP, CH * MP)], mp_v)
        pltpu.sync_copy(shk_hbm.at[pl.ds(off, CH)],
                        shk_v.at[pl.ds(0, CH)])
        pltpu.sync_copy(shp_hbm.at[pl.ds(off, CH)],
                        shp_v.at[pl.ds(0, CH)])
        cp1 = pltpu.async_copy(nidxm_hbm.at[mk_v], nidx_v, sem)
        cp2 = pltpu.async_copy(nsimm_hbm.at[mk_v], nsim_v, sem)
        cp3 = pltpu.async_copy(profm_hbm.at[mp_v], prof_v, sem)
        cp1.wait()
        cp2.wait()
        cp3.wait()

        def qgrp_body(g, carry2):
            def q_body(qi, sv):
                q = g * L + qi
                shk = shk_v[pl.ds(q, L)][0]
                shp = shp_v[pl.ds(q, L)][0]
                for gi, poff in enumerate(P_OFFS):
                    pidx = prof_v[MP * q, pl.ds(shp + poff, L)]
                    if gi == len(P_OFFS) - 1:
                        pidx = jnp.where(lane >= P_TAIL_FRESH, pidx, N_ITEMS)
                    histo_add(pidx)
                acc = zeros
                for gi, koff in enumerate(K_OFFS):
                    kidx = nidx_v[MK * q, pl.ds(shk + koff, L)]
                    ksim = nsim_v[MK * q, pl.ds(shk + koff, L)]
                    if gi == len(K_OFFS) - 1:
                        ksim = jnp.where(lane >= K_TAIL_FRESH, ksim, zeros)
                    cnt = plsc.load_gather(tbl, [kidx])
                    acc = acc + ksim * cnt
                for gi, poff in enumerate(P_OFFS):
                    pidx = prof_v[MP * q, pl.ds(shp + poff, L)]
                    if gi == len(P_OFFS) - 1:
                        pidx = jnp.where(lane >= P_TAIL_FRESH, pidx, N_ITEMS)
                    plsc.store_scatter(tbl, [pidx], zeros)
                return jnp.where(lane == qi, jnp.sum(acc), sv)
            sv = lax.fori_loop(0, L, q_body, zeros)
            score_v[pl.ds(c * CH + g * L, L)] = sv
            return carry2
        lax.fori_loop(0, CH // L, qgrp_body, 0)
        return carry
    lax.fori_loop(0, NCH, chunk_body, 0)

    pltpu.sync_copy(score_v, out_hbm.at[pl.ds(base, QW)])


_knn = functools.partial(
    pl.kernel,
    out_type=jax.ShapeDtypeStruct((B,), jnp.float32),
    mesh=plsc.VectorSubcoreMesh(core_axis_name="c", subcore_axis_name="s"),
    compiler_params=pltpu.CompilerParams(
        needs_layout_passes=False, use_tc_tiling_on_sc=False),
    scratch_types=[
        pltpu.VMEM((TBL,), jnp.float32),
        pltpu.VMEM((3 * L,), jnp.int32),
        pltpu.VMEM((CH * MK,), jnp.int32),
        pltpu.VMEM((CH * MP,), jnp.int32),
        pltpu.VMEM((CH + L,), jnp.int32),
        pltpu.VMEM((CH + L,), jnp.int32),
        pltpu.VMEM((CH * MK, L), jnp.int32),
        pltpu.VMEM((CH * MK, L), jnp.float32),
        pltpu.VMEM((CH * MP, L), jnp.int32),
        pltpu.VMEM((QW,), jnp.float32),
        pltpu.SemaphoreType.DMA,
    ],
)(_knn_body)


def kernel(u, i, nbr_idx, nbr_sim, user_prof):
    i = i.astype(jnp.int32)
    u = u.astype(jnp.int32)
    mk0 = 12 * i + i // 2
    shk = 200 * i - 16 * mk0
    mp0 = (50 * u) // 16
    shp = 50 * u - 16 * mp0
    mk = (mk0[:, None] + jnp.arange(MK, dtype=jnp.int32)).reshape(-1)
    mp = (mp0[:, None] + jnp.arange(MP, dtype=jnp.int32)).reshape(-1)
    nidxm = nbr_idx.reshape(N_ITEMS * K // L, L)
    nsimm = nbr_sim.reshape(N_ITEMS * K // L, L)
    profm = user_prof.reshape(N_USERS * P // L, L)
    return _knn(mk, shk, mp, shp, nidxm, nsimm, profm)
